# baseline (speedup 1.0000x reference)
import jax
import jax.numpy as jnp
from jax import lax
from jax.experimental import pallas as pl
from jax.experimental.pallas import tpu as pltpu

_N = 51
_NH = 4
_HID = 64
_OUT = 128
_LS = 32
_T = 50
_NC = 8
_B = 256
_G = 32


def _gat_kernel(f_ref, w1a_ref, w1b_ref, wsrc1_ref, wdst1_ref,
                wfc2_ref, wsrc2_ref, wdst2_ref, out_ref):
    w1a = w1a_ref[...]
    w1b = w1b_ref[...]
    wsrc1 = wsrc1_ref[...]
    wdst1 = wdst1_ref[...]
    wfc2 = wfc2_ref[...]
    wsrc2 = wsrc2_ref[...]
    wdst2 = wdst2_ref[...]

    ri = lax.broadcasted_iota(jnp.int32, (_N, _N), 0)
    ci = lax.broadcasted_iota(jnp.int32, (_N, _N), 1)
    diag = ri == ci

    def body(g, carry):
        h = f_ref[g]
        h0 = h[:, 0:1][None, :, :]
        h1c = h[:, 1:2][None, :, :]
        z3 = h0 * w1a + h1c * w1b

        a_src = jnp.sum(z3 * wsrc1, axis=2, keepdims=True)
        a_dst = jnp.sum(z3 * wdst1, axis=2, keepdims=True)
        e = a_src + jnp.transpose(a_dst, (0, 2, 1))
        e = jnp.where(e >= 0, e, 0.01 * e)
        e = jnp.where(diag[None, :, :], -1e9, e)
        m = jnp.max(e, axis=1, keepdims=True)
        p = jnp.exp(e - m)
        s = jnp.sum(p, axis=1, keepdims=True)
        alpha = p / s

        h1_3 = lax.dot_general(alpha, z3, (((1,), (1,)), ((0,), (0,))),
                               preferred_element_type=jnp.float32)
        h1 = jnp.concatenate([h1_3[0], h1_3[1], h1_3[2], h1_3[3]], axis=1)
        h1 = jnp.where(h1 > 0, h1, jnp.exp(jnp.minimum(h1, 0.0)) - 1.0)

        z2 = lax.dot_general(h1, wfc2, (((1,), (1,)), ((), ())),
                             preferred_element_type=jnp.float32)
        a2s = lax.dot_general(z2, wsrc2, (((1,), (1,)), ((), ())),
                              preferred_element_type=jnp.float32)
        a2d = lax.dot_general(wdst2, z2, (((1,), (1,)), ((), ())),
                              preferred_element_type=jnp.float32)
        e2 = a2s + a2d
        e2 = jnp.where(e2 >= 0, e2, 0.01 * e2)
        e2 = jnp.where(diag, -1e9, e2)
        m2 = jnp.max(e2, axis=0, keepdims=True)
        p2 = jnp.exp(e2 - m2)
        s2 = jnp.sum(p2, axis=0, keepdims=True)
        alpha2 = p2 / s2

        h2t = lax.dot_general(z2, alpha2, (((0,), (0,)), ((), ())),
                              preferred_element_type=jnp.float32)
        row = jnp.sum(h2t, axis=0, keepdims=True) * (1.0 / _OUT)
        out_ref[g] = row
        return carry

    lax.fori_loop(0, _G, body, 0)


def _lstm_kernel(ext_ref, wih0_ref, whh0_ref, b0_ref,
                 wih1_ref, whh1_ref, b1_ref, w3_ref, bout_ref,
                 out_ref, hs_ref):
    wih0 = wih0_ref[...]
    whh0 = whh0_ref[...]
    b0 = b0_ref[...]
    wih1 = wih1_ref[...]
    whh1 = whh1_ref[...]
    b1 = b1_ref[...]

    def gates(gmat, c):
        i = jax.nn.sigmoid(gmat[:, 0:_LS])
        f = jax.nn.sigmoid(gmat[:, _LS:2 * _LS])
        gg = jnp.tanh(gmat[:, 2 * _LS:3 * _LS])
        o = jax.nn.sigmoid(gmat[:, 3 * _LS:4 * _LS])
        c_new = f * c + i * gg
        h_new = o * jnp.tanh(c_new)
        return h_new, c_new

    def body(b, carry):
        h0, c0, h1, c1 = carry
        x = ext_ref[b]
        g0 = (lax.dot_general(x, wih0, (((1,), (1,)), ((), ())),
                              preferred_element_type=jnp.float32)
              + lax.dot_general(h0, whh0, (((1,), (1,)), ((), ())),
                                preferred_element_type=jnp.float32)
              + b0)
        h0n, c0n = gates(g0, c0)
        g1 = (lax.dot_general(h0n, wih1, (((1,), (1,)), ((), ())),
                              preferred_element_type=jnp.float32)
              + lax.dot_general(h1, whh1, (((1,), (1,)), ((), ())),
                                preferred_element_type=jnp.float32)
              + b1)
        h1n, c1n = gates(g1, c1)
        hs_ref[b] = h1n
        return (h0n, c0n, h1n, c1n)

    z = jnp.zeros((_T, _LS), dtype=jnp.float32)
    lax.fori_loop(0, _B, body, (z, z, z, z))

    hs = hs_ref[...]
    w3 = w3_ref[...]
    prod = lax.dot_general(hs, w3, (((2,), (2,)), ((1,), (1,))),
                           preferred_element_type=jnp.float32)
    out_ref[...] = jnp.sum(prod, axis=0) + bout_ref[...]


def kernel(features, W_fc1, W_attn1, W_fc2, W_attn2,
           w_ih0, w_hh0, b_ih0, b_hh0, w_ih1, w_hh1, b_ih1, b_hh1,
           W_out, b_out):
    f3 = features.reshape(_B * _T, _N, 2)

    w1a = W_fc1[:, :, 0].reshape(_NH, 1, _HID)
    w1b = W_fc1[:, :, 1].reshape(_NH, 1, _HID)
    wsrc1 = W_attn1[:, 0, :_HID].reshape(_NH, 1, _HID)
    wdst1 = W_attn1[:, 0, _HID:].reshape(_NH, 1, _HID)
    wsrc2 = W_attn2[:, :_OUT]
    wdst2 = W_attn2[:, _OUT:]

    ngraph = _B * _T
    grid_a = (ngraph // _G,)

    def full(shape):
        return pl.BlockSpec(shape, lambda i: tuple(0 for _ in shape))

    ext3 = pl.pallas_call(
        _gat_kernel,
        grid=grid_a,
        in_specs=[
            pl.BlockSpec((_G, _N, 2), lambda i: (i, 0, 0)),
            full(w1a.shape), full(w1b.shape), full(wsrc1.shape),
            full(wdst1.shape), full(W_fc2.shape), full(wsrc2.shape),
            full(wdst2.shape),
        ],
        out_specs=pl.BlockSpec((_G, 1, _N), lambda i: (i, 0, 0)),
        out_shape=jax.ShapeDtypeStruct((ngraph, 1, _N), jnp.float32),
    )(f3, w1a, w1b, wsrc1, wdst1, W_fc2, wsrc2, wdst2)

    ext = ext3.reshape(_B, _T, _N)

    b0 = (b_ih0 + b_hh0).reshape(1, 4 * _LS)
    b1 = (b_ih1 + b_hh1).reshape(1, 4 * _LS)
    w3 = W_out.reshape(_NC, _T, _LS)
    bout = b_out.reshape(1, _NC)

    out = pl.pallas_call(
        _lstm_kernel,
        out_shape=jax.ShapeDtypeStruct((_B, _NC), jnp.float32),
        scratch_shapes=[pltpu.VMEM((_B, _T, _LS), jnp.float32)],
    )(ext, w_ih0, w_hh0, b0, w_ih1, w_hh1, b1, w3, bout)

    return out


# speedup vs baseline: 3.6063x; 3.6063x over previous
import jax
import jax.numpy as jnp
from jax import lax
from jax.experimental import pallas as pl
from jax.experimental.pallas import tpu as pltpu

_N = 51
_NH = 4
_HID = 64
_OUT = 128
_LS = 32
_T = 50
_NC = 8
_B = 256
_G = 32


def _gat_kernel(f_ref, w1a_ref, w1b_ref, wsrc1_ref, wdst1_ref,
                wfc2_ref, wsrc2_ref, wdst2_ref, out_ref):
    w1a = w1a_ref[...]
    w1b = w1b_ref[...]
    wsrc1 = wsrc1_ref[...]
    wdst1 = wdst1_ref[...]
    wfc2 = wfc2_ref[...]
    wsrc2 = wsrc2_ref[...]
    wdst2 = wdst2_ref[...]

    ri = lax.broadcasted_iota(jnp.int32, (_N, _N), 0)
    ci = lax.broadcasted_iota(jnp.int32, (_N, _N), 1)
    diag = ri == ci

    f = f_ref[...]
    h0 = f[:, None, :, 0:1]
    h1c = f[:, None, :, 1:2]
    z4 = h0 * w1a[None] + h1c * w1b[None]

    a_src = jnp.sum(z4 * wsrc1[None], axis=3, keepdims=True)
    a_dst = jnp.sum(z4 * wdst1[None], axis=3)
    e = a_src + a_dst[:, :, None, :]
    e = jnp.where(e >= 0, e, 0.01 * e)
    e = jnp.where(diag[None, None], -1e9, e)
    m = jnp.max(e, axis=2, keepdims=True)
    p = jnp.exp(e - m)
    s = jnp.sum(p, axis=2, keepdims=True)
    alpha = p / s

    alpha_r = alpha.reshape(_G * _NH, _N, _N)
    z_r = z4.reshape(_G * _NH, _N, _HID)
    h1_3 = lax.dot_general(alpha_r, z_r, (((1,), (1,)), ((0,), (0,))),
                           preferred_element_type=jnp.float32)
    h1_4 = h1_3.reshape(_G, _NH, _N, _HID)
    h1 = jnp.concatenate([h1_4[:, 0], h1_4[:, 1], h1_4[:, 2], h1_4[:, 3]],
                         axis=2)
    h1 = jnp.where(h1 > 0, h1, jnp.exp(jnp.minimum(h1, 0.0)) - 1.0)

    z2 = lax.dot_general(h1, wfc2, (((2,), (1,)), ((), ())),
                         preferred_element_type=jnp.float32)
    a2s = jnp.sum(z2 * wsrc2[None], axis=2, keepdims=True)
    a2d = jnp.sum(z2 * wdst2[None], axis=2)
    e2 = a2s + a2d[:, None, :]
    e2 = jnp.where(e2 >= 0, e2, 0.01 * e2)
    e2 = jnp.where(diag[None], -1e9, e2)
    m2 = jnp.max(e2, axis=1, keepdims=True)
    p2 = jnp.exp(e2 - m2)
    s2 = jnp.sum(p2, axis=1, keepdims=True)
    alpha2 = p2 / s2

    h2 = lax.dot_general(alpha2, z2, (((1,), (1,)), ((0,), (0,))),
                         preferred_element_type=jnp.float32)
    out_ref[...] = jnp.sum(h2, axis=2) * (1.0 / _OUT)


def _lstm_kernel(ext_ref, wih0_ref, whh0_ref, b0_ref,
                 wih1_ref, whh1_ref, b1_ref, w3_ref, bout_ref,
                 out_ref, hs_ref):
    wih0 = wih0_ref[...]
    whh0 = whh0_ref[...]
    b0 = b0_ref[...]
    wih1 = wih1_ref[...]
    whh1 = whh1_ref[...]
    b1 = b1_ref[...]

    def gates(gmat, c):
        i = jax.nn.sigmoid(gmat[:, 0:_LS])
        f = jax.nn.sigmoid(gmat[:, _LS:2 * _LS])
        gg = jnp.tanh(gmat[:, 2 * _LS:3 * _LS])
        o = jax.nn.sigmoid(gmat[:, 3 * _LS:4 * _LS])
        c_new = f * c + i * gg
        h_new = o * jnp.tanh(c_new)
        return h_new, c_new

    def body(b, carry):
        h0, c0, h1, c1 = carry
        x = ext_ref[b]
        g0 = (lax.dot_general(x, wih0, (((1,), (1,)), ((), ())),
                              preferred_element_type=jnp.float32)
              + lax.dot_general(h0, whh0, (((1,), (1,)), ((), ())),
                                preferred_element_type=jnp.float32)
              + b0)
        h0n, c0n = gates(g0, c0)
        g1 = (lax.dot_general(h0n, wih1, (((1,), (1,)), ((), ())),
                              preferred_element_type=jnp.float32)
              + lax.dot_general(h1, whh1, (((1,), (1,)), ((), ())),
                                preferred_element_type=jnp.float32)
              + b1)
        h1n, c1n = gates(g1, c1)
        hs_ref[b] = h1n
        return (h0n, c0n, h1n, c1n)

    z = jnp.zeros((_T, _LS), dtype=jnp.float32)
    lax.fori_loop(0, _B, body, (z, z, z, z))

    hs = hs_ref[...]
    w3 = w3_ref[...]
    prod = lax.dot_general(hs, w3, (((2,), (2,)), ((1,), (1,))),
                           preferred_element_type=jnp.float32)
    out_ref[...] = jnp.sum(prod, axis=0) + bout_ref[...]


def kernel(features, W_fc1, W_attn1, W_fc2, W_attn2,
           w_ih0, w_hh0, b_ih0, b_hh0, w_ih1, w_hh1, b_ih1, b_hh1,
           W_out, b_out):
    f3 = features.reshape(_B * _T, _N, 2)

    w1a = W_fc1[:, :, 0].reshape(_NH, 1, _HID)
    w1b = W_fc1[:, :, 1].reshape(_NH, 1, _HID)
    wsrc1 = W_attn1[:, 0, :_HID].reshape(_NH, 1, _HID)
    wdst1 = W_attn1[:, 0, _HID:].reshape(_NH, 1, _HID)
    wsrc2 = W_attn2[:, :_OUT]
    wdst2 = W_attn2[:, _OUT:]

    ngraph = _B * _T
    grid_a = (ngraph // _G,)

    def full(shape):
        return pl.BlockSpec(shape, lambda i: tuple(0 for _ in shape))

    ext3 = pl.pallas_call(
        _gat_kernel,
        grid=grid_a,
        in_specs=[
            pl.BlockSpec((_G, _N, 2), lambda i: (i, 0, 0)),
            full(w1a.shape), full(w1b.shape), full(wsrc1.shape),
            full(wdst1.shape), full(W_fc2.shape), full(wsrc2.shape),
            full(wdst2.shape),
        ],
        out_specs=pl.BlockSpec((_G, _N), lambda i: (i, 0)),
        out_shape=jax.ShapeDtypeStruct((ngraph, _N), jnp.float32),
    )(f3, w1a, w1b, wsrc1, wdst1, W_fc2, wsrc2, wdst2)

    ext = ext3.reshape(_B, _T, _N)

    b0 = (b_ih0 + b_hh0).reshape(1, 4 * _LS)
    b1 = (b_ih1 + b_hh1).reshape(1, 4 * _LS)
    w3 = W_out.reshape(_NC, _T, _LS)
    bout = b_out.reshape(1, _NC)

    out = pl.pallas_call(
        _lstm_kernel,
        out_shape=jax.ShapeDtypeStruct((_B, _NC), jnp.float32),
        scratch_shapes=[pltpu.VMEM((_B, _T, _LS), jnp.float32)],
    )(ext, w_ih0, w_hh0, b0, w_ih1, w_hh1, b1, w3, bout)

    return out
